# baseline (speedup 1.0000x reference)
import functools

import jax
import jax.numpy as jnp
from jax.experimental import pallas as pl
from jax.experimental.pallas import tpu as pltpu


def _se_body(x_ref, w1t_ref, w2t_ref, o_ref, *, inv_hw):
    x = x_ref[...]

    pooled = jnp.sum(x, axis=-1, dtype=jnp.float32) * inv_hw

    h = jnp.maximum(
        jax.lax.dot(pooled, w1t_ref[...],
                    preferred_element_type=jnp.float32), 0.0)
    gate = jax.nn.sigmoid(
        jax.lax.dot(h, w2t_ref[...],
                    preferred_element_type=jnp.float32))

    o_ref[...] = x * gate[:, :, None].astype(x.dtype)


def kernel(x, w1, w2):
    B, C, H, W = x.shape
    Cr = w1.shape[0]
    HW = H * W

    xb = x.astype(jnp.bfloat16).reshape(B, C, HW)
    w1t = w1.astype(jnp.float32).T
    w2t = w2.astype(jnp.float32).T

    per_b = C * HW * 2
    bt = 1
    while bt * 2 <= B and bt * per_b < 4 * 1024 * 1024 and B % (bt * 2) == 0:
        bt *= 2
    grid = (B // bt,)

    out = pl.pallas_call(
        functools.partial(_se_body, inv_hw=1.0 / HW),
        out_shape=jax.ShapeDtypeStruct((B, C, HW), jnp.bfloat16),
        grid=grid,
        in_specs=[
            pl.BlockSpec((bt, C, HW), lambda b: (b, 0, 0)),
            pl.BlockSpec((C, Cr), lambda b: (0, 0)),
            pl.BlockSpec((Cr, C), lambda b: (0, 0)),
        ],
        out_specs=pl.BlockSpec((bt, C, HW), lambda b: (b, 0, 0)),
        compiler_params=pltpu.CompilerParams(
            dimension_semantics=("arbitrary",),
            vmem_limit_bytes=48 * 1024 * 1024,
        ),
        cost_estimate=pl.CostEstimate(
            flops=2 * B * C * HW + 4 * B * C * Cr,
            transcendentals=B * C,
            bytes_accessed=2 * B * C * HW * 2,
        ),
    )(xb, w1t, w2t)

    out4 = jax.lax.optimization_barrier(out.reshape(B, C, H, W))
    return out4.astype(jnp.float32)


# speedup vs baseline: 1.1660x; 1.1660x over previous
import functools

import jax
import jax.numpy as jnp
from jax.experimental import pallas as pl
from jax.experimental.pallas import tpu as pltpu


def _se_body(x_ref, w1t_ref, w2t_ref, o_ref, *, inv_hw):
    x = x_ref[...]

    pooled = jnp.sum(x, axis=-1, dtype=jnp.float32) * inv_hw

    h = jnp.maximum(
        jax.lax.dot(pooled, w1t_ref[...],
                    preferred_element_type=jnp.float32), 0.0)
    gate = jax.nn.sigmoid(
        jax.lax.dot(h, w2t_ref[...],
                    preferred_element_type=jnp.float32))

    o_ref[...] = x * gate[:, :, None].astype(x.dtype)


def kernel(x, w1, w2):
    B, C, H, W = x.shape
    Cr = w1.shape[0]
    HW = H * W

    xb = x.astype(jnp.bfloat16).reshape(B, C, HW)
    w1t = w1.astype(jnp.float32).T
    w2t = w2.astype(jnp.float32).T

    per_b = C * HW * 2
    bt = 1
    while bt * 2 <= B and bt * per_b < 4 * 1024 * 1024 and B % (bt * 2) == 0:
        bt *= 2
    grid = (B // bt,)

    out = pl.pallas_call(
        functools.partial(_se_body, inv_hw=1.0 / HW),
        out_shape=jax.ShapeDtypeStruct((B, C, HW), jnp.bfloat16),
        grid=grid,
        in_specs=[
            pl.BlockSpec((bt, C, HW), lambda b: (b, 0, 0)),
            pl.BlockSpec((C, Cr), lambda b: (0, 0)),
            pl.BlockSpec((Cr, C), lambda b: (0, 0)),
        ],
        out_specs=pl.BlockSpec((bt, C, HW), lambda b: (b, 0, 0)),
        compiler_params=pltpu.CompilerParams(
            dimension_semantics=("arbitrary",),
            vmem_limit_bytes=48 * 1024 * 1024,
        ),
        cost_estimate=pl.CostEstimate(
            flops=2 * B * C * HW + 4 * B * C * Cr,
            transcendentals=B * C,
            bytes_accessed=2 * B * C * HW * 2,
        ),
    )(xb, w1t, w2t)

    return out.astype(jnp.float32).reshape(B, C, H, W)
